# Initial kernel scaffold; baseline (speedup 1.0000x reference)
import jax
import jax.numpy as jnp
from jax import lax
from jax.experimental import pallas as pl
from jax.experimental.pallas import tpu as pltpu
from jax.experimental.pallas import tpu_sc as plsc

B = 4
HO = WO = 192
CC = 192
OUT_B = HO * WO * CC
IN_B = OUT_B // 4
TOTAL_OUT = B * OUT_B
NS = 16
NWIN = 2
WIN = OUT_B // (2 * NWIN)
SHARE = IN_B // NS
WSHARE = WIN // NS
CHUNK = 13824
NCHUNK = SHARE // CHUNK
GROUPS = CHUNK // 16


def _scatter_body(idx_hbm, upd_hbm, out_hbm, win_sh, idx_v, upd_v, off_v,
                  val_v, zero_v):
    c = lax.axis_index("c")
    s = lax.axis_index("s")

    def zinit(g, carry):
        zero_v[pl.ds(g * 16, 16)] = jnp.zeros((16,), jnp.float32)
        return carry

    lax.fori_loop(0, CHUNK // 16, zinit, 0)

    for b in range(B):
        for w in range(NWIN):
            wbase = c * (NWIN * WIN) + w * WIN

            for z in range(WSHARE // CHUNK):
                pltpu.sync_copy(
                    zero_v, win_sh.at[pl.ds(s * WSHARE + z * CHUNK, CHUNK)])
            plsc.subcore_barrier()

            for ch in range(NCHUNK):
                base = b * IN_B + s * SHARE + ch * CHUNK
                pltpu.sync_copy(idx_hbm.at[pl.ds(base, CHUNK)], idx_v)
                pltpu.sync_copy(upd_hbm.at[pl.ds(base, CHUNK)], upd_v)

                def body(g, carry):
                    iv = idx_v[pl.ds(g * 16, 16)]
                    uv = upd_v[pl.ds(g * 16, 16)]
                    rel = iv - wbase
                    inm = (iv >= wbase) & (rel < WIN)
                    off_v[pl.ds(g * 16, 16)] = jnp.where(inm, rel,
                                                         iv & 0xFFFF)
                    val_v[pl.ds(g * 16, 16)] = jnp.where(
                        inm, uv, jnp.zeros((16,), jnp.float32))
                    return carry

                lax.fori_loop(0, GROUPS, body, 0)
                pltpu.sync_copy(val_v, win_sh.at[off_v], add=True)
            plsc.subcore_barrier()

            out_base = b * OUT_B + wbase + s * WSHARE
            pltpu.sync_copy(win_sh.at[pl.ds(s * WSHARE, WSHARE)],
                            out_hbm.at[pl.ds(out_base, WSHARE)])
            plsc.subcore_barrier()


def kernel(updates, mask):
    idx = mask.reshape(-1).astype(jnp.int32)
    upd = updates.reshape(-1)
    mesh = plsc.VectorSubcoreMesh(core_axis_name="c", subcore_axis_name="s")
    run = pl.kernel(
        _scatter_body,
        mesh=mesh,
        out_type=jax.ShapeDtypeStruct((TOTAL_OUT,), jnp.float32),
        scratch_types=[
            pltpu.VMEM_SHARED((WIN,), jnp.float32),
            pltpu.VMEM((CHUNK,), jnp.int32),
            pltpu.VMEM((CHUNK,), jnp.float32),
            pltpu.VMEM((CHUNK,), jnp.int32),
            pltpu.VMEM((CHUNK,), jnp.float32),
            pltpu.VMEM((CHUNK,), jnp.float32),
        ],
    )
    out = run(idx, upd)
    return out.reshape(B, HO, WO, CC)


# speedup vs baseline: 18.2147x; 18.2147x over previous
import jax
import jax.numpy as jnp
from jax import lax
from jax.experimental import pallas as pl
from jax.experimental.pallas import tpu as pltpu
from jax.experimental.pallas import tpu_sc as plsc

B = 4
HO = WO = 192
CC = 192
OUT_B = HO * WO * CC
IN_B = OUT_B // 4
TOTAL_OUT = B * OUT_B
NS = 16
NWIN = 2
WIN = OUT_B // (2 * NWIN)
SHARE = IN_B // NS
WSHARE = WIN // NS
CHUNK = 4608
NCHUNK = SHARE // CHUNK
GROUPS = CHUNK // 16


def _scatter_body(idx_hbm, upd_hbm, out_hbm, win_sh, idx_v, upd_v, off_v,
                  val_v):
    c = lax.axis_index("c")
    s = lax.axis_index("s")

    for b in range(B):
        for w in range(NWIN):
            wbase = c * (NWIN * WIN) + w * WIN

            def zinit(g, carry):
                val_v[pl.ds(g * 16, 16)] = jnp.zeros((16,), jnp.float32)
                return carry

            lax.fori_loop(0, GROUPS, zinit, 0)
            for z in range(WSHARE // CHUNK):
                pltpu.sync_copy(
                    val_v, win_sh.at[pl.ds(s * WSHARE + z * CHUNK, CHUNK)])
            plsc.subcore_barrier()

            for ch in range(NCHUNK):
                base = b * IN_B + s * SHARE + ch * CHUNK
                pltpu.sync_copy(idx_hbm.at[pl.ds(base, CHUNK)], idx_v)
                pltpu.sync_copy(upd_hbm.at[pl.ds(base, CHUNK)], upd_v)

                def body(g, carry):
                    iv = idx_v[pl.ds(g * 16, 16)]
                    uv = upd_v[pl.ds(g * 16, 16)]
                    rel = iv - wbase
                    inm = (iv >= wbase) & (rel < WIN)
                    off_v[pl.ds(g * 16, 16)] = jnp.where(inm, rel,
                                                         iv & 0xFFFF)
                    val_v[pl.ds(g * 16, 16)] = jnp.where(
                        inm, uv, jnp.zeros((16,), jnp.float32))
                    return carry

                lax.fori_loop(0, GROUPS, body, 0)
                pltpu.sync_copy(val_v, win_sh.at[off_v], add=True)
            plsc.subcore_barrier()

            out_base = b * OUT_B + wbase + s * WSHARE
            pltpu.sync_copy(win_sh.at[pl.ds(s * WSHARE, WSHARE)],
                            out_hbm.at[pl.ds(out_base, WSHARE)])
            plsc.subcore_barrier()


def kernel(updates, mask):
    idx = mask.reshape(-1).astype(jnp.int32)
    upd = updates.reshape(-1)
    mesh = plsc.VectorSubcoreMesh(core_axis_name="c", subcore_axis_name="s")
    run = pl.kernel(
        _scatter_body,
        mesh=mesh,
        out_type=jax.ShapeDtypeStruct((TOTAL_OUT,), jnp.float32),
        scratch_types=[
            pltpu.VMEM_SHARED((WIN,), jnp.float32),
            pltpu.VMEM((CHUNK,), jnp.int32),
            pltpu.VMEM((CHUNK,), jnp.float32),
            pltpu.VMEM((CHUNK,), jnp.int32),
            pltpu.VMEM((CHUNK,), jnp.float32),
        ],
    )
    out = run(idx, upd)
    return out.reshape(B, HO, WO, CC)
